# Initial kernel scaffold; baseline (speedup 1.0000x reference)
import jax
import jax.numpy as jnp
from jax.experimental import pallas as pl


def kernel(x, edge_index, W, b):
    raise NotImplementedError("write your pallas kernel here")


# speedup vs baseline: 21.0408x; 21.0408x over previous
import functools

import jax
import jax.numpy as jnp
from jax import lax
from jax.experimental import pallas as pl
from jax.experimental.pallas import tpu as pltpu
from jax.experimental.pallas import tpu_sc as plsc

N = 10000
E = 320000
D = 128
C = 40
NPAD = 10240
CP = 48
NC = 2
NS = 16
NW = NC * NS
CHUNK = 128
CH = 79
EPT = CH * CHUNK
EPAD = NW * EPT
RPT = NPAD // NS

@functools.lru_cache(maxsize=None)
def _sc_kernels():
    mesh = plsc.VectorSubcoreMesh(
        core_axis_name="c", subcore_axis_name="s", num_cores=NC, num_subcores=NS
    )
    params = pltpu.CompilerParams(use_tc_tiling_on_sc=False)

    @functools.partial(
        pl.kernel,
        out_type=jax.ShapeDtypeStruct((NC, NPAD, 1), jnp.float32),
        mesh=mesh,
        scratch_types=[
            pltpu.VMEM((CH, CHUNK), jnp.int32),
            pltpu.VMEM((CHUNK, 1), jnp.float32),
            pltpu.VMEM_SHARED((NPAD, 1), jnp.float32),
        ],
        compiler_params=params,
    )
    def deg_kernel(dst_hbm, ones_hbm, zeros_hbm, out_hbm, idx_v, ones_v, deg_s):
        c = lax.axis_index("c")
        s = lax.axis_index("s")
        wid = c * NS + s
        r0 = s * RPT
        pltpu.sync_copy(zeros_hbm.at[pl.ds(r0, RPT)], deg_s.at[pl.ds(r0, RPT)])
        pltpu.sync_copy(ones_hbm, ones_v)
        pltpu.sync_copy(dst_hbm.at[wid], idx_v)
        plsc.subcore_barrier()

        def body(j, carry):
            pltpu.sync_copy(ones_v, deg_s.at[idx_v.at[j]], add=True)
            return carry

        lax.fori_loop(0, CH, body, 0)
        plsc.subcore_barrier()
        pltpu.sync_copy(deg_s.at[pl.ds(r0, RPT)], out_hbm.at[c, pl.ds(r0, RPT)])

    @functools.partial(
        pl.kernel,
        out_type=jax.ShapeDtypeStruct((NC, NPAD, CP), jnp.float32),
        mesh=mesh,
        scratch_types=[
            pltpu.VMEM((CH, CHUNK), jnp.int32),
            pltpu.VMEM((CH, CHUNK), jnp.int32),
            pltpu.VMEM((CHUNK, CP), jnp.float32),
            pltpu.VMEM_SHARED((NPAD, CP), jnp.float32),
            pltpu.SemaphoreType.DMA,
        ],
        compiler_params=params,
    )
    def hop_kernel(t_hbm, src_hbm, dst_hbm, zeros_hbm, out_hbm,
                   sidx_v, didx_v, rows_v, acc_s, sem):
        c = lax.axis_index("c")
        s = lax.axis_index("s")
        wid = c * NS + s
        r0 = s * RPT
        pltpu.sync_copy(zeros_hbm.at[pl.ds(r0, RPT)], acc_s.at[pl.ds(r0, RPT)])
        pltpu.sync_copy(src_hbm.at[wid], sidx_v)
        pltpu.sync_copy(dst_hbm.at[wid], didx_v)
        plsc.subcore_barrier()

        def body(j, carry):
            pltpu.async_copy(t_hbm.at[sidx_v.at[j]], rows_v, sem).wait()
            pltpu.sync_copy(rows_v, acc_s.at[didx_v.at[j]], add=True)
            return carry

        lax.fori_loop(0, CH, body, 0)
        plsc.subcore_barrier()
        pltpu.sync_copy(acc_s.at[pl.ds(r0, RPT)], out_hbm.at[c, pl.ds(r0, RPT)])

    return deg_kernel, hop_kernel


def _fold_body(degp, x, w, t0, dinvb, invdb):
    deg = degp[0] + degp[1] + 1.0
    dinv = lax.rsqrt(deg)
    invd = 1.0 / deg
    dinvb[...] = jnp.broadcast_to(dinv, (NPAD, CP))
    invdb[...] = jnp.broadcast_to(invd, (NPAD, CP))
    t0[...] = dinv * jnp.dot(x[...], w[...], preferred_element_type=jnp.float32)


def _comb_body(p, t0, invdb, t1):
    t1[...] = invdb[...] * (p[0] + p[1] + t0[...])


def _final_body(q, t1, dinvb, b, out):
    o = dinvb[...] * (q[0] + q[1] + t1[...]) + b[...]
    col = lax.broadcasted_iota(jnp.int32, (NPAD, CP), 1)
    om = jnp.where(col < C, o, -1e30)
    m = jnp.max(om, axis=1, keepdims=True)
    ssum = jnp.sum(jnp.exp(om - m), axis=1, keepdims=True)
    out[...] = om - m - jnp.log(ssum)


_fold = pl.pallas_call(
    _fold_body,
    out_shape=[
        jax.ShapeDtypeStruct((NPAD, CP), jnp.float32),
        jax.ShapeDtypeStruct((NPAD, CP), jnp.float32),
        jax.ShapeDtypeStruct((NPAD, CP), jnp.float32),
    ],
)

_comb = pl.pallas_call(
    _comb_body,
    out_shape=jax.ShapeDtypeStruct((NPAD, CP), jnp.float32),
)

_final = pl.pallas_call(
    _final_body,
    out_shape=jax.ShapeDtypeStruct((NPAD, CP), jnp.float32),
)


def kernel(x, edge_index, W, b):
    ei = edge_index.astype(jnp.int32)
    src = jnp.concatenate([ei[0], jnp.zeros((EPAD - E,), jnp.int32)])
    dst = jnp.concatenate([ei[1], jnp.full((EPAD - E,), NPAD - 1, jnp.int32)])
    src3 = src.reshape(NW, CH, CHUNK)
    dst3 = dst.reshape(NW, CH, CHUNK)

    x_p = jnp.zeros((NPAD, D), jnp.float32).at[:N].set(x)
    W_p = jnp.zeros((D, CP), jnp.float32).at[:, :C].set(W)
    b_p = jnp.zeros((1, CP), jnp.float32).at[0, :C].set(b)
    ones1 = jnp.ones((CHUNK, 1), jnp.float32)
    zeros_n1 = jnp.zeros((NPAD, 1), jnp.float32)
    zeros_nc = jnp.zeros((NPAD, CP), jnp.float32)

    deg_k, hop_k = _sc_kernels()
    degp = deg_k(dst3, ones1, zeros_n1)
    t0, dinvb, invdb = _fold(degp, x_p, W_p)
    p = hop_k(t0, src3, dst3, zeros_nc)
    t1 = _comb(p, t0, invdb)
    q = hop_k(t1, src3, dst3, zeros_nc)
    out = _final(q, t1, dinvb, b_p)
    return out[:N, :C]


# speedup vs baseline: 26.2805x; 1.2490x over previous
import functools

import jax
import jax.numpy as jnp
from jax import lax
from jax.experimental import pallas as pl
from jax.experimental.pallas import tpu as pltpu
from jax.experimental.pallas import tpu_sc as plsc

N = 10000
E = 320000
D = 128
C = 40
NPAD = 10240
CP = 48
NC = 2
NS = 16
NW = NC * NS
CHUNK = 128
CH = 79
EPT = CH * CHUNK
EPAD = NW * EPT
RPT = NPAD // NS

@functools.lru_cache(maxsize=None)
def _sc_kernels():
    mesh = plsc.VectorSubcoreMesh(
        core_axis_name="c", subcore_axis_name="s", num_cores=NC, num_subcores=NS
    )
    params = pltpu.CompilerParams(use_tc_tiling_on_sc=False)

    @functools.partial(
        pl.kernel,
        out_type=jax.ShapeDtypeStruct((NC, NPAD, 1), jnp.float32),
        mesh=mesh,
        scratch_types=[
            pltpu.VMEM((CH, CHUNK), jnp.int32),
            pltpu.VMEM((CHUNK, 1), jnp.float32),
            pltpu.VMEM_SHARED((NPAD, 1), jnp.float32),
        ],
        compiler_params=params,
    )
    def deg_kernel(dst_hbm, ones_hbm, zeros_hbm, out_hbm, idx_v, ones_v, deg_s):
        c = lax.axis_index("c")
        s = lax.axis_index("s")
        wid = c * NS + s
        r0 = s * RPT
        pltpu.sync_copy(zeros_hbm.at[pl.ds(r0, RPT)], deg_s.at[pl.ds(r0, RPT)])
        pltpu.sync_copy(ones_hbm, ones_v)
        pltpu.sync_copy(dst_hbm.at[wid], idx_v)
        plsc.subcore_barrier()

        def body(j, carry):
            pltpu.sync_copy(ones_v, deg_s.at[idx_v.at[j]], add=True)
            return carry

        lax.fori_loop(0, CH, body, 0)
        plsc.subcore_barrier()
        pltpu.sync_copy(deg_s.at[pl.ds(r0, RPT)], out_hbm.at[c, pl.ds(r0, RPT)])

    @functools.partial(
        pl.kernel,
        out_type=jax.ShapeDtypeStruct((NC, NPAD, CP), jnp.float32),
        mesh=mesh,
        scratch_types=[
            pltpu.VMEM((CH, CHUNK), jnp.int32),
            pltpu.VMEM((CH, CHUNK), jnp.int32),
            pltpu.VMEM((2, CHUNK, CP), jnp.float32),
            pltpu.VMEM_SHARED((NPAD, CP), jnp.float32),
            pltpu.SemaphoreType.DMA((2,)),
            pltpu.SemaphoreType.DMA((2,)),
        ],
        compiler_params=params,
    )
    def hop_kernel(t_hbm, src_hbm, dst_hbm, zeros_hbm, out_hbm,
                   sidx_v, didx_v, rows_v, acc_s, gsem, ssem):
        c = lax.axis_index("c")
        s = lax.axis_index("s")
        wid = c * NS + s
        r0 = s * RPT
        pltpu.sync_copy(zeros_hbm.at[pl.ds(r0, RPT)], acc_s.at[pl.ds(r0, RPT)])
        pltpu.sync_copy(src_hbm.at[wid], sidx_v)
        pltpu.sync_copy(dst_hbm.at[wid], didx_v)
        plsc.subcore_barrier()

        pltpu.async_copy(t_hbm.at[sidx_v.at[0]], rows_v.at[0], gsem.at[0])

        def body(j, carry):
            p = j % 2
            pn = (j + 1) % 2

            @pl.when(j + 1 < CH)
            def _():
                pltpu.async_copy(
                    t_hbm.at[sidx_v.at[j + 1]], rows_v.at[pn], gsem.at[pn]
                )

            pltpu.make_async_copy(
                t_hbm.at[sidx_v.at[j]], rows_v.at[p], gsem.at[p]
            ).wait()
            pltpu.sync_copy(rows_v.at[p], acc_s.at[didx_v.at[j]], add=True)
            return carry

        lax.fori_loop(0, CH, body, 0)
        plsc.subcore_barrier()
        pltpu.sync_copy(acc_s.at[pl.ds(r0, RPT)], out_hbm.at[c, pl.ds(r0, RPT)])

    return deg_kernel, hop_kernel


def _fold_body(degp, x, w, t0, dinvb, invdb):
    deg = degp[0] + degp[1] + 1.0
    dinv = lax.rsqrt(deg)
    invd = 1.0 / deg
    dinvb[...] = jnp.broadcast_to(dinv, (NPAD, CP))
    invdb[...] = jnp.broadcast_to(invd, (NPAD, CP))
    t0[...] = dinv * jnp.dot(x[...], w[...], preferred_element_type=jnp.float32)


def _comb_body(p, t0, invdb, t1):
    t1[...] = invdb[...] * (p[0] + p[1] + t0[...])


def _final_body(q, t1, dinvb, b, out):
    o = dinvb[...] * (q[0] + q[1] + t1[...]) + b[...]
    col = lax.broadcasted_iota(jnp.int32, (NPAD, CP), 1)
    om = jnp.where(col < C, o, -1e30)
    m = jnp.max(om, axis=1, keepdims=True)
    ssum = jnp.sum(jnp.exp(om - m), axis=1, keepdims=True)
    out[...] = om - m - jnp.log(ssum)


_fold = pl.pallas_call(
    _fold_body,
    out_shape=[
        jax.ShapeDtypeStruct((NPAD, CP), jnp.float32),
        jax.ShapeDtypeStruct((NPAD, CP), jnp.float32),
        jax.ShapeDtypeStruct((NPAD, CP), jnp.float32),
    ],
)

_comb = pl.pallas_call(
    _comb_body,
    out_shape=jax.ShapeDtypeStruct((NPAD, CP), jnp.float32),
)

_final = pl.pallas_call(
    _final_body,
    out_shape=jax.ShapeDtypeStruct((NPAD, CP), jnp.float32),
)


def kernel(x, edge_index, W, b):
    ei = edge_index.astype(jnp.int32)
    src = jnp.concatenate([ei[0], jnp.zeros((EPAD - E,), jnp.int32)])
    dst = jnp.concatenate([ei[1], jnp.full((EPAD - E,), NPAD - 1, jnp.int32)])
    src3 = src.reshape(NW, CH, CHUNK)
    dst3 = dst.reshape(NW, CH, CHUNK)

    x_p = jnp.zeros((NPAD, D), jnp.float32).at[:N].set(x)
    W_p = jnp.zeros((D, CP), jnp.float32).at[:, :C].set(W)
    b_p = jnp.zeros((1, CP), jnp.float32).at[0, :C].set(b)
    ones1 = jnp.ones((CHUNK, 1), jnp.float32)
    zeros_n1 = jnp.zeros((NPAD, 1), jnp.float32)
    zeros_nc = jnp.zeros((NPAD, CP), jnp.float32)

    deg_k, hop_k = _sc_kernels()
    degp = deg_k(dst3, ones1, zeros_n1)
    t0, dinvb, invdb = _fold(degp, x_p, W_p)
    p = hop_k(t0, src3, dst3, zeros_nc)
    t1 = _comb(p, t0, invdb)
    q = hop_k(t1, src3, dst3, zeros_nc)
    out = _final(q, t1, dinvb, b_p)
    return out[:N, :C]
